# baseline (speedup 1.0000x reference)
import jax
import jax.numpy as jnp
from jax.experimental import pallas as pl
from jax.experimental.pallas import tpu as pltpu

E = 8
H = 1024
I = 2048
T = 32

BI = 512
NJ = I // BI
NSTEPS = E * NJ
NSH = NSTEPS // 2
BIS = I // NSH


def _silu(x):
    return x * jax.nn.sigmoid(x)


def _moe_body(x_ref, rw_ref, gate_ref, up_ref, down_ref,
              shg_ref, shu_ref, shd_ref,
              out_ref, scores_ref, sc_scratch):
    k = pl.program_id(0)

    @pl.when(k == 0)
    def _init():
        x = x_ref[...]
        logits = jax.lax.dot_general(
            x, rw_ref[...], (((1,), (1,)), ((), ())),
            preferred_element_type=jnp.float32)
        idx = jnp.argmax(logits, axis=1)
        sig = jax.nn.sigmoid(logits)
        eids = jax.lax.broadcasted_iota(jnp.int32, (T, E), 1)
        sc = jnp.where(eids == idx[:, None], sig, 0.0)
        scT = sc.T
        sc_scratch[...] = scT
        scores_ref[...] = scT
        out_ref[...] = jnp.zeros_like(out_ref)

    e = k // NJ
    srow = sc_scratch[pl.ds(e, 1), :]
    xs = x_ref[...] * srow.T
    g = jnp.dot(xs, gate_ref[0], preferred_element_type=jnp.float32)
    u = jnp.dot(xs, up_ref[0], preferred_element_type=jnp.float32)
    a = u * _silu(g)
    acc = jnp.dot(a, down_ref[0], preferred_element_type=jnp.float32)

    out_ref[...] += acc

    @pl.when(k % 2 == 0)
    def _shared():
        x = x_ref[...]
        gs = jax.lax.dot_general(x, shg_ref[...], (((1,), (1,)), ((), ())),
                                 preferred_element_type=jnp.float32)
        us = jax.lax.dot_general(x, shu_ref[...], (((1,), (1,)), ((), ())),
                                 preferred_element_type=jnp.float32)
        as_ = _silu(gs) * us
        out_ref[...] += jax.lax.dot_general(
            as_, shd_ref[...], (((1,), (1,)), ((), ())),
            preferred_element_type=jnp.float32)


def kernel(hidden_states, router_w, gate_up_proj, down_proj,
           sh_gate, sh_up, sh_down):
    x = hidden_states.reshape(-1, H)

    out, scores = pl.pallas_call(
        _moe_body,
        grid=(NSTEPS,),
        in_specs=[
            pl.BlockSpec((T, H), lambda k: (0, 0)),
            pl.BlockSpec((E, H), lambda k: (0, 0)),
            pl.BlockSpec((1, H, BI), lambda k: (k // NJ, 0, k % NJ)),
            pl.BlockSpec((1, H, BI), lambda k: (k // NJ, 0, NJ + k % NJ)),
            pl.BlockSpec((1, BI, H), lambda k: (k // NJ, k % NJ, 0)),
            pl.BlockSpec((BIS, H), lambda k: (k // 2, 0)),
            pl.BlockSpec((BIS, H), lambda k: (k // 2, 0)),
            pl.BlockSpec((H, BIS), lambda k: (0, k // 2)),
        ],
        out_specs=[
            pl.BlockSpec((T, H), lambda k: (0, 0)),
            pl.BlockSpec((E, T), lambda k: (0, 0)),
        ],
        out_shape=[
            jax.ShapeDtypeStruct((T, H), jnp.float32),
            jax.ShapeDtypeStruct((E, T), jnp.float32),
        ],
        scratch_shapes=[pltpu.VMEM((E, T), jnp.float32)],
        compiler_params=pltpu.CompilerParams(
            dimension_semantics=("arbitrary",),
        ),
    )(x, router_w, gate_up_proj, gate_up_proj, down_proj,
      sh_gate, sh_up, sh_down)

    return (out, scores)


# speedup vs baseline: 1.1246x; 1.1246x over previous
import jax
import jax.numpy as jnp
from jax.experimental import pallas as pl
from jax.experimental.pallas import tpu as pltpu

E = 8
H = 1024
I = 2048
T = 32

BI = 1024
NJ = I // BI
NSTEPS = E * NJ
BIS = I // NSTEPS


def _silu(x):
    return x * jax.nn.sigmoid(x)


def _moe_body(x_ref, rw_ref, gate_ref, up_ref, down_ref,
              shg_ref, shu_ref, shd_ref,
              out_ref, scores_ref, sc_scratch):
    k = pl.program_id(0)

    @pl.when(k == 0)
    def _init():
        x = x_ref[...]
        logits = jax.lax.dot_general(
            x, rw_ref[...], (((1,), (1,)), ((), ())),
            preferred_element_type=jnp.float32)
        idx = jnp.argmax(logits, axis=1)
        sig = jax.nn.sigmoid(logits)
        eids = jax.lax.broadcasted_iota(jnp.int32, (T, E), 1)
        sc = jnp.where(eids == idx[:, None], sig, 0.0)
        scT = sc.T
        sc_scratch[...] = scT
        scores_ref[...] = scT
        out_ref[...] = jnp.zeros_like(out_ref)

    e = k // NJ
    srow = sc_scratch[pl.ds(e, 1), :]
    xs = x_ref[...] * srow.T
    g = jnp.dot(xs, gate_ref[0], preferred_element_type=jnp.float32)
    u = jnp.dot(xs, up_ref[0], preferred_element_type=jnp.float32)
    a = u * _silu(g)
    acc = jnp.dot(a, down_ref[0], preferred_element_type=jnp.float32)

    x = x_ref[...]
    gs = jax.lax.dot_general(x, shg_ref[...], (((1,), (1,)), ((), ())),
                             preferred_element_type=jnp.float32)
    us = jax.lax.dot_general(x, shu_ref[...], (((1,), (1,)), ((), ())),
                             preferred_element_type=jnp.float32)
    as_ = _silu(gs) * us
    acc += jax.lax.dot_general(as_, shd_ref[...], (((1,), (1,)), ((), ())),
                               preferred_element_type=jnp.float32)

    out_ref[...] += acc


def kernel(hidden_states, router_w, gate_up_proj, down_proj,
           sh_gate, sh_up, sh_down):
    x = hidden_states.reshape(-1, H)

    out, scores = pl.pallas_call(
        _moe_body,
        grid=(NSTEPS,),
        in_specs=[
            pl.BlockSpec((T, H), lambda k: (0, 0)),
            pl.BlockSpec((E, H), lambda k: (0, 0)),
            pl.BlockSpec((1, H, BI), lambda k: (k // NJ, 0, k % NJ)),
            pl.BlockSpec((1, H, BI), lambda k: (k // NJ, 0, NJ + k % NJ)),
            pl.BlockSpec((1, BI, H), lambda k: (k // NJ, k % NJ, 0)),
            pl.BlockSpec((BIS, H), lambda k: (k, 0)),
            pl.BlockSpec((BIS, H), lambda k: (k, 0)),
            pl.BlockSpec((H, BIS), lambda k: (0, k)),
        ],
        out_specs=[
            pl.BlockSpec((T, H), lambda k: (0, 0)),
            pl.BlockSpec((E, T), lambda k: (0, 0)),
        ],
        out_shape=[
            jax.ShapeDtypeStruct((T, H), jnp.float32),
            jax.ShapeDtypeStruct((E, T), jnp.float32),
        ],
        scratch_shapes=[pltpu.VMEM((E, T), jnp.float32)],
        compiler_params=pltpu.CompilerParams(
            dimension_semantics=("arbitrary",),
        ),
    )(x, router_w, gate_up_proj, gate_up_proj, down_proj,
      sh_gate, sh_up, sh_down)

    return (out, scores)
